# baseline (speedup 1.0000x reference)
import functools

import jax
import jax.numpy as jnp
from jax import lax
from jax.experimental import pallas as pl
from jax.experimental.pallas import tpu as pltpu
from jax.experimental.pallas import tpu_sc as plsc

N_NODES = 10000
D_FEAT = 128
N_EDGES = 320000

NC, NS, L = 2, 16, 16
W_ROW = D_FEAT // 2
NW = NC * NS
E_W = N_EDGES // NW
C = 80
NCHUNK = E_W // C
CPB = 25
BLOCK = C * CPB


def _body(h_hbm, src_hbm, dst_hbm, out_hbm,
          h_sh, idx_s, idx_d, rows_s, rows_d, out_v, sem_e, sem_h, sem_o):
    sid = lax.axis_index("s")
    wid = sid * NC + lax.axis_index("c")
    base_w = wid * E_W

    @pl.when(sid < 10)
    def _stage_h():
        rs = pl.ds(sid * (N_NODES // 10), N_NODES // 10)
        pltpu.sync_copy(h_hbm.at[rs], h_sh.at[rs])

    def refill(blk):
        bs = pl.ds(base_w + blk * BLOCK, BLOCK)
        pltpu.sync_copy(src_hbm.at[bs], idx_s)
        pltpu.sync_copy(dst_hbm.at[bs], idx_d)

    refill(0)
    plsc.subcore_barrier()

    def fire(c):
        b = c & 1
        sl = pl.ds((c % CPB) * C, C)

        @pl.when(b == 0)
        def _from_spmem():
            pltpu.async_copy(h_sh.at[idx_s.at[sl]], rows_s.at[b], sem_e)
            pltpu.async_copy(h_sh.at[idx_d.at[sl]], rows_d.at[b], sem_e)

        @pl.when(b == 1)
        def _from_hbm():
            pltpu.async_copy(h_hbm.at[idx_s.at[sl]], rows_s.at[b], sem_h)
            pltpu.async_copy(h_hbm.at[idx_d.at[sl]], rows_d.at[b], sem_h)

    def drain(c):
        b = c & 1
        sl = pl.ds((c % CPB) * C, C)

        @pl.when(b == 0)
        def _from_spmem():
            pltpu.make_async_copy(
                h_sh.at[idx_s.at[sl]], rows_s.at[b], sem_e).wait()
            pltpu.make_async_copy(
                h_sh.at[idx_d.at[sl]], rows_d.at[b], sem_e).wait()

        @pl.when(b == 1)
        def _from_hbm():
            pltpu.make_async_copy(
                h_hbm.at[idx_s.at[sl]], rows_s.at[b], sem_h).wait()
            pltpu.make_async_copy(
                h_hbm.at[idx_d.at[sl]], rows_d.at[b], sem_h).wait()

    fire(0)
    fire(1)
    lane = lax.iota(jnp.int32, L)

    def chunk_body(c, carry):
        b = c & 1

        @pl.when(c % CPB != CPB - 1)
        def _drain_self():
            drain(c)

        @pl.when(c >= 2)
        def _drain_out():
            pltpu.make_async_copy(
                out_v.at[b], out_hbm.at[pl.ds(base_w + (c - 2) * C, C)], sem_o
            ).wait()

        def group_body(g, carry2):
            def edge_body(j, r):
                e = g * L + j
                a = jnp.zeros((L,), jnp.float32)
                for k in range(W_ROW // L):
                    bs = plsc.bitcast(rows_s[b, e, pl.ds(k * L, L)],
                                      jnp.bfloat16)
                    bd = plsc.bitcast(rows_d[b, e, pl.ds(k * L, L)],
                                      jnp.bfloat16)
                    u0, u1 = plsc.unpack(bs * bd,
                                         format=plsc.PackFormat.INTERLEAVED)
                    a = a + u0 + u1
                return jnp.where(lane == j, jnp.sum(a), r)

            r = lax.fori_loop(0, L, edge_body, jnp.zeros((L,), jnp.float32),
                              unroll=4)
            out_v[b, pl.ds(g * L, L)] = 1.0 / (1.0 + jnp.exp(-r))
            return carry2

        lax.fori_loop(0, C // L, group_body, 0)

        @pl.when((c + 2 < NCHUNK) & ((c + 2) % CPB == 0))
        def _refill_next():
            drain(c + 1)
            refill((c + 2) // CPB)

        @pl.when(c + 2 < NCHUNK)
        def _fire_next():
            fire(c + 2)

        pltpu.async_copy(out_v.at[b], out_hbm.at[pl.ds(base_w + c * C, C)],
                         sem_o)
        return carry

    lax.fori_loop(0, NCHUNK, chunk_body, 0)

    for cc in (NCHUNK - 2, NCHUNK - 1):
        pltpu.make_async_copy(
            out_v.at[cc & 1], out_hbm.at[pl.ds(base_w + cc * C, C)], sem_o
        ).wait()


_mesh = plsc.VectorSubcoreMesh(core_axis_name="c", subcore_axis_name="s")

_decoder = pl.kernel(
    _body,
    out_type=jax.ShapeDtypeStruct((N_EDGES,), jnp.float32),
    mesh=_mesh,
    scratch_types=[
        pltpu.VMEM_SHARED((N_NODES, D_FEAT), jnp.int32),
        pltpu.VMEM((BLOCK,), jnp.int32),
        pltpu.VMEM((BLOCK,), jnp.int32),
        pltpu.VMEM((2, C, D_FEAT), jnp.int32),
        pltpu.VMEM((2, C, D_FEAT), jnp.int32),
        pltpu.VMEM((2, C), jnp.float32),
        pltpu.SemaphoreType.DMA,
        pltpu.SemaphoreType.DMA,
        pltpu.SemaphoreType.DMA,
    ],
    compiler_params=pltpu.CompilerParams(needs_layout_passes=False),
)


@jax.jit
def kernel(h, edge_index):
    src = edge_index[0]
    dst = edge_index[1]
    h_packed = jax.lax.bitcast_convert_type(
        h.astype(jnp.bfloat16).reshape(N_NODES, W_ROW, 2), jnp.int32
    )
    h_padded = jnp.concatenate(
        [h_packed, jnp.zeros((N_NODES, D_FEAT - W_ROW), jnp.int32)], axis=1
    )
    return _decoder(h_padded, src, dst)


# speedup vs baseline: 1.1159x; 1.1159x over previous
import functools

import jax
import jax.numpy as jnp
from jax import lax
from jax.experimental import pallas as pl
from jax.experimental.pallas import tpu as pltpu
from jax.experimental.pallas import tpu_sc as plsc

N_NODES = 10000
D_FEAT = 128
N_EDGES = 320000

NC, NS, L = 2, 16, 16
W_ROW = D_FEAT // 2
NW = NC * NS
E_W = N_EDGES // NW
C = 80
NCHUNK = E_W // C
CPB = 25
BLOCK = C * CPB
NBLK = NCHUNK // CPB


def _body(h_hbm, src_hbm, dst_hbm, out_hbm,
          h_sh, idx_s, idx_d, rows_s, rows_d, out_v, sem_g, sem_o):
    sid = lax.axis_index("s")
    wid = sid * NC + lax.axis_index("c")
    base_w = wid * E_W

    @pl.when(sid < 10)
    def _stage_h():
        rs = pl.ds(sid * (N_NODES // 10), N_NODES // 10)
        pltpu.sync_copy(h_hbm.at[rs], h_sh.at[rs])

    def refill(blk):
        bs = pl.ds(base_w + blk * BLOCK, BLOCK)
        pltpu.sync_copy(src_hbm.at[bs], idx_s)
        pltpu.sync_copy(dst_hbm.at[bs], idx_d)

    refill(0)
    plsc.subcore_barrier()

    def fire(c):
        b = c & 1
        sl = pl.ds((c % CPB) * C, C)
        pltpu.async_copy(h_sh.at[idx_s.at[sl]], rows_s.at[b], sem_g)
        pltpu.async_copy(h_sh.at[idx_d.at[sl]], rows_d.at[b], sem_g)

    fire(0)
    lane = lax.iota(jnp.int32, L)

    def chunk_body(c, carry):
        b = c & 1

        sl = pl.ds((c % CPB) * C, C)
        pltpu.make_async_copy(h_sh.at[idx_s.at[sl]], rows_s.at[b], sem_g).wait()
        pltpu.make_async_copy(h_sh.at[idx_d.at[sl]], rows_d.at[b], sem_g).wait()

        @pl.when((c + 1 < NCHUNK) & ((c + 1) % CPB == 0))
        def _refill_next():
            refill((c + 1) // CPB)

        @pl.when(c + 1 < NCHUNK)
        def _fire_next():
            fire(c + 1)

        @pl.when(c >= 2)
        def _drain_out():
            pltpu.make_async_copy(
                out_v.at[b], out_hbm.at[pl.ds(base_w + (c - 2) * C, C)], sem_o
            ).wait()

        def group_body(g, carry2):
            def edge_body(j, r):
                e = g * L + j
                a = jnp.zeros((L,), jnp.float32)
                for k in range(W_ROW // L):
                    bs = plsc.bitcast(rows_s[b, e, pl.ds(k * L, L)],
                                      jnp.bfloat16)
                    bd = plsc.bitcast(rows_d[b, e, pl.ds(k * L, L)],
                                      jnp.bfloat16)
                    u0, u1 = plsc.unpack(bs * bd,
                                         format=plsc.PackFormat.INTERLEAVED)
                    a = a + u0 + u1
                return jnp.where(lane == j, jnp.sum(a), r)

            r = lax.fori_loop(0, L, edge_body, jnp.zeros((L,), jnp.float32),
                              unroll=4)
            out_v[b, pl.ds(g * L, L)] = 1.0 / (1.0 + jnp.exp(-r))
            return carry2

        lax.fori_loop(0, C // L, group_body, 0)
        pltpu.async_copy(out_v.at[b], out_hbm.at[pl.ds(base_w + c * C, C)],
                         sem_o)
        return carry

    lax.fori_loop(0, NCHUNK, chunk_body, 0)

    for cc in (NCHUNK - 2, NCHUNK - 1):
        pltpu.make_async_copy(
            out_v.at[cc & 1], out_hbm.at[pl.ds(base_w + cc * C, C)], sem_o
        ).wait()


_mesh = plsc.VectorSubcoreMesh(core_axis_name="c", subcore_axis_name="s")

_decoder = pl.kernel(
    _body,
    out_type=jax.ShapeDtypeStruct((N_EDGES,), jnp.float32),
    mesh=_mesh,
    scratch_types=[
        pltpu.VMEM_SHARED((N_NODES, D_FEAT), jnp.int32),
        pltpu.VMEM((BLOCK,), jnp.int32),
        pltpu.VMEM((BLOCK,), jnp.int32),
        pltpu.VMEM((2, C, D_FEAT), jnp.int32),
        pltpu.VMEM((2, C, D_FEAT), jnp.int32),
        pltpu.VMEM((2, C), jnp.float32),
        pltpu.SemaphoreType.DMA,
        pltpu.SemaphoreType.DMA,
    ],
    compiler_params=pltpu.CompilerParams(needs_layout_passes=False),
)


@jax.jit
def kernel(h, edge_index):
    src = edge_index[0]
    dst = edge_index[1]
    h_packed = jax.lax.bitcast_convert_type(
        h.astype(jnp.bfloat16).reshape(N_NODES, W_ROW, 2), jnp.int32
    )
    h_padded = jnp.concatenate(
        [h_packed, jnp.zeros((N_NODES, D_FEAT - W_ROW), jnp.int32)], axis=1
    )
    return _decoder(h_padded, src, dst)


# speedup vs baseline: 1.1417x; 1.0232x over previous
import functools

import jax
import jax.numpy as jnp
from jax import lax
from jax.experimental import pallas as pl
from jax.experimental.pallas import tpu as pltpu
from jax.experimental.pallas import tpu_sc as plsc

N_NODES = 10000
D_FEAT = 128
N_EDGES = 320000

NC, NS, L = 2, 16, 16
NW = NC * NS
E_W = N_EDGES // NW
C = 80
NCHUNK = E_W // C
CPB = 25
BLOCK = C * CPB
NBLK = NCHUNK // CPB


def _body(h_hbm, src_hbm, dst_hbm, out_hbm,
          h_sh, idx_s, idx_d, rows_s, rows_d, out_v, sem_g, sem_h, sem_o):
    sid = lax.axis_index("s")
    wid = sid * NC + lax.axis_index("c")
    base_w = wid * E_W

    @pl.when(sid < 10)
    def _stage_h():
        rs = pl.ds(sid * (N_NODES // 10), N_NODES // 10)
        pltpu.sync_copy(h_hbm.at[rs], h_sh.at[rs])

    def refill(blk):
        bs = pl.ds(base_w + blk * BLOCK, BLOCK)
        pltpu.sync_copy(src_hbm.at[bs], idx_s)
        pltpu.sync_copy(dst_hbm.at[bs], idx_d)

    refill(0)
    plsc.subcore_barrier()

    def fire(c):
        b = c & 1
        sl = pl.ds((c % CPB) * C, C)
        pltpu.async_copy(h_sh.at[idx_s.at[sl]], rows_s.at[b], sem_g)
        pltpu.async_copy(h_hbm.at[idx_d.at[sl]], rows_d.at[b], sem_h)

    fire(0)
    lane = lax.iota(jnp.int32, L)

    def chunk_body(c, carry):
        b = c & 1

        sl = pl.ds((c % CPB) * C, C)
        pltpu.make_async_copy(h_sh.at[idx_s.at[sl]], rows_s.at[b], sem_g).wait()
        pltpu.make_async_copy(h_hbm.at[idx_d.at[sl]], rows_d.at[b], sem_h).wait()

        @pl.when((c + 1 < NCHUNK) & ((c + 1) % CPB == 0))
        def _refill_next():
            refill((c + 1) // CPB)

        @pl.when(c + 1 < NCHUNK)
        def _fire_next():
            fire(c + 1)

        @pl.when(c >= 2)
        def _drain_out():
            pltpu.make_async_copy(
                out_v.at[b], out_hbm.at[pl.ds(base_w + (c - 2) * C, C)], sem_o
            ).wait()

        def group_body(g, carry2):
            def edge_body(j, r):
                e = g * L + j
                a = rows_s[b, e, pl.ds(0, L)] * rows_d[b, e, pl.ds(0, L)]
                for k in range(1, D_FEAT // L):
                    a = a + (rows_s[b, e, pl.ds(k * L, L)]
                             * rows_d[b, e, pl.ds(k * L, L)])
                return jnp.where(lane == j, jnp.sum(a), r)

            r = lax.fori_loop(0, L, edge_body, jnp.zeros((L,), jnp.float32),
                              unroll=4)
            out_v[b, pl.ds(g * L, L)] = 1.0 / (1.0 + jnp.exp(-r))
            return carry2

        lax.fori_loop(0, C // L, group_body, 0)
        pltpu.async_copy(out_v.at[b], out_hbm.at[pl.ds(base_w + c * C, C)],
                         sem_o)
        return carry

    lax.fori_loop(0, NCHUNK, chunk_body, 0)

    for cc in (NCHUNK - 2, NCHUNK - 1):
        pltpu.make_async_copy(
            out_v.at[cc & 1], out_hbm.at[pl.ds(base_w + cc * C, C)], sem_o
        ).wait()


_mesh = plsc.VectorSubcoreMesh(core_axis_name="c", subcore_axis_name="s")

_decoder = pl.kernel(
    _body,
    out_type=jax.ShapeDtypeStruct((N_EDGES,), jnp.float32),
    mesh=_mesh,
    scratch_types=[
        pltpu.VMEM_SHARED((N_NODES, D_FEAT), jnp.float32),
        pltpu.VMEM((BLOCK,), jnp.int32),
        pltpu.VMEM((BLOCK,), jnp.int32),
        pltpu.VMEM((2, C, D_FEAT), jnp.float32),
        pltpu.VMEM((2, C, D_FEAT), jnp.float32),
        pltpu.VMEM((2, C), jnp.float32),
        pltpu.SemaphoreType.DMA,
        pltpu.SemaphoreType.DMA,
        pltpu.SemaphoreType.DMA,
    ],
    compiler_params=pltpu.CompilerParams(needs_layout_passes=False),
)


@jax.jit
def kernel(h, edge_index):
    src = edge_index[0]
    dst = edge_index[1]
    return _decoder(h, src, dst)


# speedup vs baseline: 1.3163x; 1.1529x over previous
import functools

import jax
import jax.numpy as jnp
from jax import lax
from jax.experimental import pallas as pl
from jax.experimental.pallas import tpu as pltpu
from jax.experimental.pallas import tpu_sc as plsc

N_NODES = 10000
D_FEAT = 128
N_EDGES = 320000

NC, NS, L = 2, 16, 16
NW = NC * NS
E_W = N_EDGES // NW
C = 80
NCHUNK = E_W // C
CPB = 25
BLOCK = C * CPB
NBLK = NCHUNK // CPB


def _body(h_hbm, src_hbm, dst_hbm, out_hbm,
          h_sh, idx_s, idx_d, rows_s, rows_d, out_v, sem_g, sem_o):
    sid = lax.axis_index("s")
    wid = sid * NC + lax.axis_index("c")
    base_w = wid * E_W

    @pl.when(sid < 10)
    def _stage_h():
        rs = pl.ds(sid * (N_NODES // 10), N_NODES // 10)
        pltpu.sync_copy(h_hbm.at[rs], h_sh.at[rs])

    def refill(blk):
        bs = pl.ds(base_w + blk * BLOCK, BLOCK)
        pltpu.sync_copy(src_hbm.at[bs], idx_s)
        pltpu.sync_copy(dst_hbm.at[bs], idx_d)

    refill(0)
    plsc.subcore_barrier()

    def fire(c):
        b = c & 1
        sl = pl.ds((c % CPB) * C, C)
        pltpu.async_copy(h_sh.at[idx_s.at[sl]], rows_s.at[b], sem_g)
        pltpu.async_copy(h_sh.at[idx_d.at[sl]], rows_d.at[b], sem_g)

    fire(0)
    lane = lax.iota(jnp.int32, L)

    def chunk_body(c, carry):
        b = c & 1

        sl = pl.ds((c % CPB) * C, C)
        pltpu.make_async_copy(h_sh.at[idx_s.at[sl]], rows_s.at[b], sem_g).wait()
        pltpu.make_async_copy(h_sh.at[idx_d.at[sl]], rows_d.at[b], sem_g).wait()

        @pl.when((c + 1 < NCHUNK) & ((c + 1) % CPB == 0))
        def _refill_next():
            refill((c + 1) // CPB)

        @pl.when(c + 1 < NCHUNK)
        def _fire_next():
            fire(c + 1)

        @pl.when(c >= 2)
        def _drain_out():
            pltpu.make_async_copy(
                out_v.at[b], out_hbm.at[pl.ds(base_w + (c - 2) * C, C)], sem_o
            ).wait()

        def group_body(g, carry2):
            def edge_body(j, r):
                e = g * L + j
                a = rows_s[b, e, pl.ds(0, L)] * rows_d[b, e, pl.ds(0, L)]
                for k in range(1, D_FEAT // L):
                    a = a + (rows_s[b, e, pl.ds(k * L, L)]
                             * rows_d[b, e, pl.ds(k * L, L)])
                return jnp.where(lane == j, jnp.sum(a), r)

            r = lax.fori_loop(0, L, edge_body, jnp.zeros((L,), jnp.float32),
                              unroll=4)
            out_v[b, pl.ds(g * L, L)] = 1.0 / (1.0 + jnp.exp(-r))
            return carry2

        lax.fori_loop(0, C // L, group_body, 0)
        pltpu.async_copy(out_v.at[b], out_hbm.at[pl.ds(base_w + c * C, C)],
                         sem_o)
        return carry

    lax.fori_loop(0, NCHUNK, chunk_body, 0)

    for cc in (NCHUNK - 2, NCHUNK - 1):
        pltpu.make_async_copy(
            out_v.at[cc & 1], out_hbm.at[pl.ds(base_w + cc * C, C)], sem_o
        ).wait()


_mesh = plsc.VectorSubcoreMesh(core_axis_name="c", subcore_axis_name="s")

_decoder = pl.kernel(
    _body,
    out_type=jax.ShapeDtypeStruct((N_EDGES,), jnp.float32),
    mesh=_mesh,
    scratch_types=[
        pltpu.VMEM_SHARED((N_NODES, D_FEAT), jnp.float32),
        pltpu.VMEM((BLOCK,), jnp.int32),
        pltpu.VMEM((BLOCK,), jnp.int32),
        pltpu.VMEM((2, C, D_FEAT), jnp.float32),
        pltpu.VMEM((2, C, D_FEAT), jnp.float32),
        pltpu.VMEM((2, C), jnp.float32),
        pltpu.SemaphoreType.DMA,
        pltpu.SemaphoreType.DMA,
    ],
    compiler_params=pltpu.CompilerParams(needs_layout_passes=False),
)


@jax.jit
def kernel(h, edge_index):
    src = edge_index[0]
    dst = edge_index[1]
    return _decoder(h, src, dst)
